# baseline (device time: 17950 ns/iter reference)
import jax
import jax.numpy as jnp
from jax import lax
from jax.experimental import pallas as pl
from jax.experimental.pallas import tpu as pltpu

N_DEV = 4


def kernel(x, W1, W2):
    m, k = x.shape
    n = W2.shape[1]
    mc = m // N_DEV

    def body(
        x_ref,
        w1_ref,
        w2_ref,
        out_ref,
        rs_send_ref,
        rs_recv_ref,
        ag_send_ref,
        ag_recv_ref,
        rs_send_sems,
        rs_recv_sems,
        ag_send_sems,
        ag_recv_sems,
    ):
        my = lax.axis_index("i")

        barrier_sem = pltpu.get_barrier_semaphore()
        for o in range(1, N_DEV):
            pl.semaphore_signal(
                barrier_sem,
                inc=1,
                device_id=((my + o) % N_DEV,),
                device_id_type=pl.DeviceIdType.MESH,
            )

        def compute_chunk(j):
            h = jnp.maximum(
                jnp.dot(
                    x_ref[pl.ds(j * mc, mc), :],
                    w1_ref[...],
                    preferred_element_type=jnp.float32,
                ),
                0.0,
            )
            return jnp.dot(h, w2_ref[...], preferred_element_type=jnp.float32)

        NSUB = 4
        mh = mc // NSUB

        def rs_rdma(slot, h, target):
            return pltpu.make_async_remote_copy(
                src_ref=rs_send_ref.at[slot, pl.ds(h * mh, mh), :],
                dst_ref=rs_recv_ref.at[slot, pl.ds(h * mh, mh), :],
                send_sem=rs_send_sems.at[slot, h],
                recv_sem=rs_recv_sems.at[slot, h],
                device_id=(target,),
                device_id_type=pl.DeviceIdType.MESH,
            )

        def ag_rdma(slot, h, target):
            return pltpu.make_async_remote_copy(
                src_ref=ag_send_ref.at[pl.ds(h * mh, mh), :],
                dst_ref=ag_recv_ref.at[slot, pl.ds(h * mh, mh), :],
                send_sem=ag_send_sems.at[slot, h],
                recv_sem=ag_recv_sems.at[slot, h],
                device_id=(target,),
                device_id_type=pl.DeviceIdType.MESH,
            )

        rs_sends = []
        for idx, o in enumerate((2, 1, 3)):
            j = (my + o) % N_DEV
            slot = N_DEV - 1 - o
            rs_send_ref[slot] = compute_chunk(j).astype(jnp.bfloat16)
            if idx == 0:
                pl.semaphore_wait(barrier_sem, N_DEV - 1)
            for h in range(NSUB):
                rdma = rs_rdma(slot, h, j)
                rdma.start()
                rs_sends.append(rdma)

        own = compute_chunk(my)

        ag_sends = []
        for h in range(NSUB):
            reduced = own[h * mh:(h + 1) * mh, :]
            for s in (1, 2, 0):
                rs_rdma(s, h, my).wait_recv()
                reduced = reduced + rs_recv_ref[s, pl.ds(h * mh, mh), :].astype(
                    jnp.float32
                )
            out_ref[pl.ds(my * mc + h * mh, mh), :] = reduced
            ag_send_ref[pl.ds(h * mh, mh), :] = reduced.astype(jnp.bfloat16)
            for o in (2, 1, 3):
                rdma = ag_rdma(N_DEV - 1 - o, h, (my + o) % N_DEV)
                rdma.start()
                ag_sends.append(rdma)

        for h in range(NSUB):
            for s in (0, 2, 1):
                i_src = (my + s + 1) % N_DEV
                ag_rdma(s, h, my).wait_recv()
                out_ref[pl.ds(i_src * mc + h * mh, mh), :] = ag_recv_ref[
                    s, pl.ds(h * mh, mh), :
                ].astype(jnp.float32)

        for rdma in rs_sends:
            rdma.wait_send()
        for rdma in ag_sends:
            rdma.wait_send()

    return pl.pallas_call(
        body,
        out_shape=jax.ShapeDtypeStruct((m, n), jnp.float32),
        in_specs=[
            pl.BlockSpec(memory_space=pltpu.VMEM),
            pl.BlockSpec(memory_space=pltpu.VMEM),
            pl.BlockSpec(memory_space=pltpu.VMEM),
        ],
        out_specs=pl.BlockSpec(memory_space=pltpu.VMEM),
        scratch_shapes=[
            pltpu.VMEM((N_DEV - 1, mc, n), jnp.bfloat16),
            pltpu.VMEM((N_DEV - 1, mc, n), jnp.bfloat16),
            pltpu.VMEM((mc, n), jnp.bfloat16),
            pltpu.VMEM((N_DEV - 1, mc, n), jnp.bfloat16),
            pltpu.SemaphoreType.DMA((N_DEV - 1, 4)),
            pltpu.SemaphoreType.DMA((N_DEV - 1, 4)),
            pltpu.SemaphoreType.DMA((N_DEV - 1, 4)),
            pltpu.SemaphoreType.DMA((N_DEV - 1, 4)),
        ],
        compiler_params=pltpu.CompilerParams(collective_id=0),
    )(x, W1, W2)


# device time: 17686 ns/iter; 1.0149x vs baseline; 1.0149x over previous
import jax
import jax.numpy as jnp
from jax import lax
from jax.experimental import pallas as pl
from jax.experimental.pallas import tpu as pltpu

N_DEV = 4


def kernel(x, W1, W2):
    m, k = x.shape
    n = W2.shape[1]
    mc = m // N_DEV

    def body(
        x_ref,
        w1_ref,
        w2_ref,
        out_ref,
        rs_send_ref,
        rs_recv_ref,
        ag_send_ref,
        ag_recv_ref,
        rs_send_sems,
        rs_recv_sems,
        ag_send_sems,
        ag_recv_sems,
    ):
        my = lax.axis_index("i")

        barrier_sem = pltpu.get_barrier_semaphore()
        for o in range(1, N_DEV):
            pl.semaphore_signal(
                barrier_sem,
                inc=1,
                device_id=((my + o) % N_DEV,),
                device_id_type=pl.DeviceIdType.MESH,
            )

        def compute_chunk(j):
            h = jnp.maximum(
                jnp.dot(
                    x_ref[pl.ds(j * mc, mc), :],
                    w1_ref[...],
                    preferred_element_type=jnp.float32,
                ),
                0.0,
            )
            return jnp.dot(h, w2_ref[...], preferred_element_type=jnp.float32)

        mh = mc // 2

        def rs_rdma(slot, h, target):
            return pltpu.make_async_remote_copy(
                src_ref=rs_send_ref.at[slot, pl.ds(h * mh, mh), :],
                dst_ref=rs_recv_ref.at[slot, pl.ds(h * mh, mh), :],
                send_sem=rs_send_sems.at[slot, h],
                recv_sem=rs_recv_sems.at[slot, h],
                device_id=(target,),
                device_id_type=pl.DeviceIdType.MESH,
            )

        def ag_rdma(slot, h, target):
            return pltpu.make_async_remote_copy(
                src_ref=ag_send_ref.at[pl.ds(h * mh, mh), :],
                dst_ref=ag_recv_ref.at[slot, pl.ds(h * mh, mh), :],
                send_sem=ag_send_sems.at[slot, h],
                recv_sem=ag_recv_sems.at[slot, h],
                device_id=(target,),
                device_id_type=pl.DeviceIdType.MESH,
            )

        rs_sends = []
        for idx, o in enumerate((2, 1, 3)):
            j = (my + o) % N_DEV
            slot = N_DEV - 1 - o
            rs_send_ref[slot] = compute_chunk(j).astype(jnp.bfloat16)
            if idx == 0:
                pl.semaphore_wait(barrier_sem, N_DEV - 1)
            for h in (0, 1):
                rdma = rs_rdma(slot, h, j)
                rdma.start()
                rs_sends.append(rdma)

        own = compute_chunk(my)

        ag_sends = []
        for h in (0, 1):
            reduced = own[h * mh:(h + 1) * mh, :]
            for s in (1, 2, 0):
                rs_rdma(s, h, my).wait_recv()
                reduced = reduced + rs_recv_ref[s, pl.ds(h * mh, mh), :].astype(
                    jnp.float32
                )
            out_ref[pl.ds(my * mc + h * mh, mh), :] = reduced
            ag_send_ref[pl.ds(h * mh, mh), :] = reduced.astype(jnp.bfloat16)
            for o in (2, 1, 3):
                rdma = ag_rdma(N_DEV - 1 - o, h, (my + o) % N_DEV)
                rdma.start()
                ag_sends.append(rdma)

        for h in (0, 1):
            for s in (0, 2, 1):
                i_src = (my + s + 1) % N_DEV
                ag_rdma(s, h, my).wait_recv()
                out_ref[pl.ds(i_src * mc + h * mh, mh), :] = ag_recv_ref[
                    s, pl.ds(h * mh, mh), :
                ].astype(jnp.float32)

        for rdma in rs_sends:
            rdma.wait_send()
        for rdma in ag_sends:
            rdma.wait_send()

    return pl.pallas_call(
        body,
        out_shape=jax.ShapeDtypeStruct((m, n), jnp.float32),
        in_specs=[
            pl.BlockSpec(memory_space=pltpu.VMEM),
            pl.BlockSpec(memory_space=pltpu.VMEM),
            pl.BlockSpec(memory_space=pltpu.VMEM),
        ],
        out_specs=pl.BlockSpec(memory_space=pltpu.VMEM),
        scratch_shapes=[
            pltpu.VMEM((N_DEV - 1, mc, n), jnp.bfloat16),
            pltpu.VMEM((N_DEV - 1, mc, n), jnp.bfloat16),
            pltpu.VMEM((mc, n), jnp.bfloat16),
            pltpu.VMEM((N_DEV - 1, mc, n), jnp.bfloat16),
            pltpu.SemaphoreType.DMA((N_DEV - 1, 2)),
            pltpu.SemaphoreType.DMA((N_DEV - 1, 2)),
            pltpu.SemaphoreType.DMA((N_DEV - 1, 2)),
            pltpu.SemaphoreType.DMA((N_DEV - 1, 2)),
        ],
        compiler_params=pltpu.CompilerParams(collective_id=0),
    )(x, W1, W2)
